# baseline (speedup 1.0000x reference)
import functools

import jax
import jax.numpy as jnp
from jax import lax
from jax.experimental import pallas as pl
from jax.experimental.pallas import tpu as pltpu
from jax.experimental.pallas import tpu_sc as plsc

N_NODES = 10000
N_PAD = 10240
H = 16
NW = 32
N_CHUNK = 80
CH = 128
E_PAD = NW * N_CHUNK * CH
ROWS_PER_TILE = N_PAD // 16
NP8 = N_PAD // 8

DEG_WIN = 16


def _deg_body(dst_hbm, out_hbm, idx_v, ones_v, buf_v, buf16_v, acc_sh, dsem):
    cid = lax.axis_index("c")
    sid = lax.axis_index("s")
    wid = cid * 16 + sid

    def _fill_ones(i, _):
        ones_v[pl.ds(i * 16, 16)] = jnp.ones((16,), jnp.float32)
        return 0

    lax.fori_loop(0, CH // 16, _fill_ones, 0)

    def _fill_zeros(i, _):
        buf_v[pl.ds(i * 16, 16)] = jnp.zeros((16,), jnp.float32)
        return 0

    lax.fori_loop(0, ROWS_PER_TILE // 16, _fill_zeros, 0)
    pltpu.sync_copy(buf_v, acc_sh.at[pl.ds(sid * ROWS_PER_TILE, ROWS_PER_TILE)])
    pltpu.sync_copy(dst_hbm.at[wid], idx_v)
    plsc.subcore_barrier()

    def _edges(j, _):
        @pl.when(j >= DEG_WIN)
        def _absorb():
            pltpu.make_async_copy(ones_v, acc_sh.at[idx_v.at[0]], dsem).wait()

        pltpu.async_copy(ones_v, acc_sh.at[idx_v.at[j]], dsem, add=True)
        return 0

    lax.fori_loop(0, N_CHUNK, _edges, 0)
    for _ in range(DEG_WIN):
        pltpu.make_async_copy(ones_v, acc_sh.at[idx_v.at[0]], dsem).wait()
    plsc.subcore_barrier()
    pltpu.sync_copy(acc_sh.at[pl.ds(sid * ROWS_PER_TILE, ROWS_PER_TILE)], buf_v)

    def _bcast(i, _):
        v = buf_v[pl.ds(i * 16, 16)]
        for l in range(16):
            buf16_v[i * 16 + l, :] = jnp.full((H,), v[l], jnp.float32)
        return 0

    lax.fori_loop(0, ROWS_PER_TILE // 16, _bcast, 0)
    pltpu.sync_copy(buf16_v, out_hbm.at[cid].at[pl.ds(sid * ROWS_PER_TILE, ROWS_PER_TILE)])


NSLOT = 10
GLOOK = 5


def _edge_body(g_hbm, src_hbm, dst_hbm, out_hbm, src_v, dst_v, rows_v, buf_v, acc_sh,
               g_sh, isem, *sems):
    gsems = sems[:NSLOT]
    ssems = sems[NSLOT:]
    cid = lax.axis_index("c")
    sid = lax.axis_index("s")
    wid = cid * 16 + sid

    pltpu.async_copy(src_hbm.at[wid], src_v, isem)
    pltpu.async_copy(dst_hbm.at[wid], dst_v, isem)

    def _fill_zeros(i, _):
        buf_v[i, :] = jnp.zeros((H,), jnp.float32)
        return 0

    lax.fori_loop(0, ROWS_PER_TILE, _fill_zeros, 0)
    pltpu.sync_copy(buf_v, acc_sh.at[pl.ds(sid * ROWS_PER_TILE, ROWS_PER_TILE)])
    pltpu.sync_copy(g_hbm.at[pl.ds(sid * ROWS_PER_TILE, ROWS_PER_TILE)], buf_v)
    pltpu.sync_copy(buf_v, g_sh.at[pl.ds(sid * ROWS_PER_TILE, ROWS_PER_TILE)])
    pltpu.make_async_copy(src_hbm.at[wid], src_v, isem).wait()
    pltpu.make_async_copy(dst_hbm.at[wid], dst_v, isem).wait()
    plsc.subcore_barrier()

    for b in range(GLOOK):
        pltpu.async_copy(g_sh.at[src_v.at[b]], rows_v.at[b], gsems[b])

    def _rounds(o, _):
        for b in range(NSLOT):
            j = o * NSLOT + b
            bg = (b + GLOOK) % NSLOT

            @pl.when(j + GLOOK < N_CHUNK)
            def _refill():
                @pl.when(j >= NSLOT - GLOOK)
                def _absorb():
                    pltpu.make_async_copy(
                        rows_v.at[0], acc_sh.at[dst_v.at[0]], ssems[bg]).wait()

                pltpu.async_copy(g_sh.at[src_v.at[j + GLOOK]], rows_v.at[bg], gsems[bg])

            pltpu.make_async_copy(g_sh.at[src_v.at[j]], rows_v.at[b], gsems[b]).wait()
            pltpu.async_copy(rows_v.at[b], acc_sh.at[dst_v.at[j]], ssems[b], add=True)
        return 0

    lax.fori_loop(0, N_CHUNK // NSLOT, _rounds, 0)
    for b in range(NSLOT):
        pltpu.make_async_copy(rows_v.at[0], acc_sh.at[dst_v.at[0]], ssems[b]).wait()
    plsc.subcore_barrier()
    pltpu.sync_copy(acc_sh.at[pl.ds(sid * ROWS_PER_TILE, ROWS_PER_TILE)], buf_v)
    pltpu.sync_copy(buf_v, out_hbm.at[cid].at[pl.ds(sid * ROWS_PER_TILE, ROWS_PER_TILE)])


@functools.cache
def _sc_kernels():
    mesh = plsc.VectorSubcoreMesh(core_axis_name="c", subcore_axis_name="s")
    params = pltpu.CompilerParams(use_tc_tiling_on_sc=False)
    deg_kernel = pl.kernel(
        _deg_body,
        out_type=jax.ShapeDtypeStruct((2, N_PAD, H), jnp.float32),
        mesh=mesh,
        compiler_params=params,
        scratch_types=[
            pltpu.VMEM((N_CHUNK, CH), jnp.int32),
            pltpu.VMEM((CH,), jnp.float32),
            pltpu.VMEM((ROWS_PER_TILE,), jnp.float32),
            pltpu.VMEM((ROWS_PER_TILE, H), jnp.float32),
            pltpu.VMEM_SHARED((N_PAD,), jnp.float32),
            pltpu.SemaphoreType.DMA,
        ],
    )
    edge_kernel = pl.kernel(
        _edge_body,
        out_type=jax.ShapeDtypeStruct((2, N_PAD, H), jnp.float32),
        mesh=mesh,
        compiler_params=params,
        scratch_types=[
            pltpu.VMEM((N_CHUNK, CH), jnp.int32),
            pltpu.VMEM((N_CHUNK, CH), jnp.int32),
            pltpu.VMEM((NSLOT, CH, H), jnp.float32),
            pltpu.VMEM((ROWS_PER_TILE, H), jnp.float32),
            pltpu.VMEM_SHARED((N_PAD, H), jnp.float32),
            pltpu.VMEM_SHARED((N_PAD, H), jnp.float32),
            pltpu.SemaphoreType.DMA,
        ] + [pltpu.SemaphoreType.DMA] * (2 * NSLOT),
    )
    return deg_kernel, edge_kernel


def _mmh_body(x_ref, w_ref, o_ref):
    h = jnp.dot(x_ref[...], w_ref[...], preferred_element_type=jnp.float32)
    jrow = lax.broadcasted_iota(jnp.int32, (H, 128), 0)
    lcol = lax.broadcasted_iota(jnp.int32, (H, 128), 1)
    acc = jnp.zeros((NP8, 128), jnp.float32)
    for b in range(8):
        if (b + 1) * NP8 <= N_NODES:
            blk = lax.slice(h, (b * NP8, 0), ((b + 1) * NP8, H))
        else:
            blk = jnp.concatenate(
                [lax.slice(h, (b * NP8, 0), (N_NODES, H)),
                 jnp.zeros(((b + 1) * NP8 - N_NODES, H), jnp.float32)], axis=0)
        sel = (lcol == H * b + jrow).astype(jnp.float32)
        acc = acc + jnp.dot(blk, sel, preferred_element_type=jnp.float32)
    o_ref[...] = acc


def _scale_body(deg_ref, h_ref, g_ref, dinv_ref):
    d = deg_ref[...]
    dinv = lax.rsqrt(d[0] + d[1] + 1.0)
    dinv_ref[...] = dinv
    g_ref[...] = dinv * h_ref[...]


def _layer2_body(dinv_ref, pre_ref, g1_ref, b1_ref, w2_ref, o_ref):
    dinv = dinv_ref[...]
    p = pre_ref[...]
    out1 = jnp.maximum(dinv * (g1_ref[...] + p[0] + p[1]) + b1_ref[...], 0.0)
    o_ref[...] = dinv * jnp.dot(out1, w2_ref[...], preferred_element_type=jnp.float32)


def _final_body(dinv_ref, pre_ref, g2_ref, b2_ref, o_ref):
    p = pre_ref[...]
    logits = dinv_ref[...] * (g2_ref[...] + p[0] + p[1]) + b2_ref[...]
    col = lax.broadcasted_iota(jnp.int32, (NP8, 128), 1)
    xm = jnp.where(col % H < 10, logits, jnp.float32(-1e30))
    m = jnp.max(xm, axis=1, keepdims=True)
    e = jnp.where(col % H < 10, jnp.exp(logits - m), 0.0)
    r = lax.broadcasted_iota(jnp.int32, (128, 128), 0)
    c = lax.broadcasted_iota(jnp.int32, (128, 128), 1)
    ones_bd = (r // H == c // H).astype(jnp.float32)
    s = jnp.dot(e, ones_bd, preferred_element_type=jnp.float32)
    o_ref[...] = (logits - m) - jnp.log(s)


_mmh = pl.pallas_call(_mmh_body, out_shape=jax.ShapeDtypeStruct((NP8, 128), jnp.float32))
_scale = pl.pallas_call(_scale_body, out_shape=(
    jax.ShapeDtypeStruct((NP8, 128), jnp.float32),
    jax.ShapeDtypeStruct((NP8, 128), jnp.float32)))
_layer2 = pl.pallas_call(_layer2_body, out_shape=jax.ShapeDtypeStruct((NP8, 128), jnp.float32))
_final = pl.pallas_call(_final_body, out_shape=jax.ShapeDtypeStruct((NP8, 128), jnp.float32))


def kernel(x, edge_index, W1, b1, W2, b2):
    n, f = x.shape
    pad_e = E_PAD - edge_index.shape[1]
    dummy = jnp.broadcast_to(
        (N_NODES + jnp.arange(pad_e, dtype=jnp.int32) % (N_PAD - N_NODES))
        .reshape(pad_e // CH, CH), (2, pad_e // CH, CH))
    ep = jnp.concatenate([edge_index.reshape(2, -1, CH), dummy], axis=1)
    ep = (ep % NP8) * 8 + ep // NP8
    src_p = ep[0].reshape(NW, N_CHUNK, CH)
    dst_p = ep[1].reshape(NW, N_CHUNK, CH)
    w2p = jnp.concatenate([W2, jnp.zeros((W2.shape[0], H - W2.shape[1]), W2.dtype)], axis=1)
    w2bd = jax.scipy.linalg.block_diag(*([w2p] * 8))
    b1t = jnp.tile(b1, 8).reshape(1, 128)
    b2p = jnp.concatenate([b2, jnp.zeros((H - b2.shape[0],), b2.dtype)])
    b2t = jnp.tile(b2p, 8).reshape(1, 128)

    _deg_kernel, _edge_kernel = _sc_kernels()
    hp = _mmh(x, W1)
    deg16 = _deg_kernel(dst_p)
    g1p, dinvp = _scale(deg16.reshape(2, NP8, 128), hp)
    pre1 = _edge_kernel(g1p.reshape(N_PAD, H), src_p, dst_p)
    g2p = _layer2(dinvp, pre1.reshape(2, NP8, 128), g1p, b1t, w2bd)
    pre2 = _edge_kernel(g2p.reshape(N_PAD, H), src_p, dst_p)
    outp = _final(dinvp, pre2.reshape(2, NP8, 128), g2p, b2t)
    out = outp.reshape(NP8, 8, H).transpose(1, 0, 2).reshape(N_PAD, H)
    return out[:n, :10]


# speedup vs baseline: 1.0937x; 1.0937x over previous
import functools

import jax
import jax.numpy as jnp
from jax import lax
from jax.experimental import pallas as pl
from jax.experimental.pallas import tpu as pltpu
from jax.experimental.pallas import tpu_sc as plsc

N_NODES = 10000
N_PAD = 10240
H = 16
NW = 32
N_CHUNK = 80
CH = 128
E_PAD = NW * N_CHUNK * CH
ROWS_PER_TILE = N_PAD // 16
NP8 = N_PAD // 8

DEG_WIN = 16


def _deg_body(dst_hbm, out_hbm, idx_v, ones_v, buf_v, buf16_v, acc_sh, dsem):
    cid = lax.axis_index("c")
    sid = lax.axis_index("s")
    wid = cid * 16 + sid

    def _fill_ones(i, _):
        ones_v[pl.ds(i * 16, 16)] = jnp.ones((16,), jnp.float32)
        return 0

    lax.fori_loop(0, CH // 16, _fill_ones, 0)

    def _fill_zeros(i, _):
        buf_v[pl.ds(i * 16, 16)] = jnp.zeros((16,), jnp.float32)
        return 0

    lax.fori_loop(0, ROWS_PER_TILE // 16, _fill_zeros, 0)
    pltpu.sync_copy(buf_v, acc_sh.at[pl.ds(sid * ROWS_PER_TILE, ROWS_PER_TILE)])
    pltpu.sync_copy(dst_hbm.at[wid], idx_v)
    plsc.subcore_barrier()

    def _edges(j, _):
        @pl.when(j >= DEG_WIN)
        def _absorb():
            pltpu.make_async_copy(ones_v, acc_sh.at[idx_v.at[0]], dsem).wait()

        pltpu.async_copy(ones_v, acc_sh.at[idx_v.at[j]], dsem, add=True)
        return 0

    lax.fori_loop(0, N_CHUNK, _edges, 0)
    for _ in range(DEG_WIN):
        pltpu.make_async_copy(ones_v, acc_sh.at[idx_v.at[0]], dsem).wait()
    plsc.subcore_barrier()
    pltpu.sync_copy(acc_sh.at[pl.ds(sid * ROWS_PER_TILE, ROWS_PER_TILE)], buf_v)

    def _bcast(i, _):
        v = buf_v[pl.ds(i * 16, 16)]
        for l in range(16):
            buf16_v[i * 16 + l, :] = jnp.full((H,), v[l], jnp.float32)
        return 0

    lax.fori_loop(0, ROWS_PER_TILE // 16, _bcast, 0)
    pltpu.sync_copy(buf16_v, out_hbm.at[cid].at[pl.ds(sid * ROWS_PER_TILE, ROWS_PER_TILE)])


NSLOT = 10
GLOOK = 5


def _edge_body(g_hbm, src_hbm, dst_hbm, out_hbm, src_v, dst_v, rows_v, buf_v, acc_sh,
               isem, *sems):
    gsems = sems[:NSLOT]
    ssems = sems[NSLOT:]
    cid = lax.axis_index("c")
    sid = lax.axis_index("s")
    wid = cid * 16 + sid

    pltpu.async_copy(src_hbm.at[wid], src_v, isem)
    pltpu.async_copy(dst_hbm.at[wid], dst_v, isem)

    def _fill_zeros(i, _):
        buf_v[i, :] = jnp.zeros((H,), jnp.float32)
        return 0

    lax.fori_loop(0, ROWS_PER_TILE, _fill_zeros, 0)
    pltpu.sync_copy(buf_v, acc_sh.at[pl.ds(sid * ROWS_PER_TILE, ROWS_PER_TILE)])
    pltpu.make_async_copy(src_hbm.at[wid], src_v, isem).wait()
    pltpu.make_async_copy(dst_hbm.at[wid], dst_v, isem).wait()
    plsc.subcore_barrier()

    for b in range(GLOOK):
        pltpu.async_copy(g_hbm.at[src_v.at[b]], rows_v.at[b], gsems[b])

    def _rounds(o, _):
        for b in range(NSLOT):
            j = o * NSLOT + b
            bg = (b + GLOOK) % NSLOT

            @pl.when(j + GLOOK < N_CHUNK)
            def _refill():
                @pl.when(j >= NSLOT - GLOOK)
                def _absorb():
                    pltpu.make_async_copy(
                        rows_v.at[0], acc_sh.at[dst_v.at[0]], ssems[bg]).wait()

                pltpu.async_copy(g_hbm.at[src_v.at[j + GLOOK]], rows_v.at[bg], gsems[bg])

            pltpu.make_async_copy(g_hbm.at[src_v.at[j]], rows_v.at[b], gsems[b]).wait()
            pltpu.async_copy(rows_v.at[b], acc_sh.at[dst_v.at[j]], ssems[b], add=True)
        return 0

    lax.fori_loop(0, N_CHUNK // NSLOT, _rounds, 0)
    for b in range(NSLOT):
        pltpu.make_async_copy(rows_v.at[0], acc_sh.at[dst_v.at[0]], ssems[b]).wait()
    plsc.subcore_barrier()
    pltpu.sync_copy(acc_sh.at[pl.ds(sid * ROWS_PER_TILE, ROWS_PER_TILE)], buf_v)
    pltpu.sync_copy(buf_v, out_hbm.at[cid].at[pl.ds(sid * ROWS_PER_TILE, ROWS_PER_TILE)])


@functools.cache
def _sc_kernels():
    mesh = plsc.VectorSubcoreMesh(core_axis_name="c", subcore_axis_name="s")
    params = pltpu.CompilerParams(use_tc_tiling_on_sc=False)
    deg_kernel = pl.kernel(
        _deg_body,
        out_type=jax.ShapeDtypeStruct((2, N_PAD, H), jnp.float32),
        mesh=mesh,
        compiler_params=params,
        scratch_types=[
            pltpu.VMEM((N_CHUNK, CH), jnp.int32),
            pltpu.VMEM((CH,), jnp.float32),
            pltpu.VMEM((ROWS_PER_TILE,), jnp.float32),
            pltpu.VMEM((ROWS_PER_TILE, H), jnp.float32),
            pltpu.VMEM_SHARED((N_PAD,), jnp.float32),
            pltpu.SemaphoreType.DMA,
        ],
    )
    edge_kernel = pl.kernel(
        _edge_body,
        out_type=jax.ShapeDtypeStruct((2, N_PAD, H), jnp.float32),
        mesh=mesh,
        compiler_params=params,
        scratch_types=[
            pltpu.VMEM((N_CHUNK, CH), jnp.int32),
            pltpu.VMEM((N_CHUNK, CH), jnp.int32),
            pltpu.VMEM((NSLOT, CH, H), jnp.float32),
            pltpu.VMEM((ROWS_PER_TILE, H), jnp.float32),
            pltpu.VMEM_SHARED((N_PAD, H), jnp.float32),
            pltpu.SemaphoreType.DMA,
        ] + [pltpu.SemaphoreType.DMA] * (2 * NSLOT),
    )
    return deg_kernel, edge_kernel


def _mmh_body(x_ref, w_ref, o_ref):
    h = jnp.dot(x_ref[...], w_ref[...], preferred_element_type=jnp.float32)
    jrow = lax.broadcasted_iota(jnp.int32, (H, 128), 0)
    lcol = lax.broadcasted_iota(jnp.int32, (H, 128), 1)
    acc = jnp.zeros((NP8, 128), jnp.float32)
    for b in range(8):
        if (b + 1) * NP8 <= N_NODES:
            blk = lax.slice(h, (b * NP8, 0), ((b + 1) * NP8, H))
        else:
            blk = jnp.concatenate(
                [lax.slice(h, (b * NP8, 0), (N_NODES, H)),
                 jnp.zeros(((b + 1) * NP8 - N_NODES, H), jnp.float32)], axis=0)
        sel = (lcol == H * b + jrow).astype(jnp.float32)
        acc = acc + jnp.dot(blk, sel, preferred_element_type=jnp.float32)
    o_ref[...] = acc


def _scale_body(deg_ref, h_ref, g_ref, dinv_ref):
    d = deg_ref[...]
    dinv = lax.rsqrt(d[0] + d[1] + 1.0)
    dinv_ref[...] = dinv
    g_ref[...] = dinv * h_ref[...]


def _layer2_body(dinv_ref, pre_ref, g1_ref, b1_ref, w2_ref, o_ref):
    dinv = dinv_ref[...]
    p = pre_ref[...]
    out1 = jnp.maximum(dinv * (g1_ref[...] + p[0] + p[1]) + b1_ref[...], 0.0)
    o_ref[...] = dinv * jnp.dot(out1, w2_ref[...], preferred_element_type=jnp.float32)


def _final_body(dinv_ref, pre_ref, g2_ref, b2_ref, o_ref):
    p = pre_ref[...]
    logits = dinv_ref[...] * (g2_ref[...] + p[0] + p[1]) + b2_ref[...]
    col = lax.broadcasted_iota(jnp.int32, (NP8, 128), 1)
    xm = jnp.where(col % H < 10, logits, jnp.float32(-1e30))
    m = jnp.max(xm, axis=1, keepdims=True)
    e = jnp.where(col % H < 10, jnp.exp(logits - m), 0.0)
    r = lax.broadcasted_iota(jnp.int32, (128, 128), 0)
    c = lax.broadcasted_iota(jnp.int32, (128, 128), 1)
    ones_bd = (r // H == c // H).astype(jnp.float32)
    s = jnp.dot(e, ones_bd, preferred_element_type=jnp.float32)
    o_ref[...] = (logits - m) - jnp.log(s)


_mmh = pl.pallas_call(_mmh_body, out_shape=jax.ShapeDtypeStruct((NP8, 128), jnp.float32))
_scale = pl.pallas_call(_scale_body, out_shape=(
    jax.ShapeDtypeStruct((NP8, 128), jnp.float32),
    jax.ShapeDtypeStruct((NP8, 128), jnp.float32)))
_layer2 = pl.pallas_call(_layer2_body, out_shape=jax.ShapeDtypeStruct((NP8, 128), jnp.float32))
_final = pl.pallas_call(_final_body, out_shape=jax.ShapeDtypeStruct((NP8, 128), jnp.float32))


def kernel(x, edge_index, W1, b1, W2, b2):
    n, f = x.shape
    pad_e = E_PAD - edge_index.shape[1]
    dummy = jnp.broadcast_to(
        (N_NODES + jnp.arange(pad_e, dtype=jnp.int32) % (N_PAD - N_NODES))
        .reshape(pad_e // CH, CH), (2, pad_e // CH, CH))
    ep = jnp.concatenate([edge_index.reshape(2, -1, CH), dummy], axis=1)
    ep = (ep % NP8) * 8 + ep // NP8
    src_p = ep[0].reshape(NW, N_CHUNK, CH)
    dst_p = ep[1].reshape(NW, N_CHUNK, CH)
    w2p = jnp.concatenate([W2, jnp.zeros((W2.shape[0], H - W2.shape[1]), W2.dtype)], axis=1)
    w2bd = jax.scipy.linalg.block_diag(*([w2p] * 8))
    b1t = jnp.tile(b1, 8).reshape(1, 128)
    b2p = jnp.concatenate([b2, jnp.zeros((H - b2.shape[0],), b2.dtype)])
    b2t = jnp.tile(b2p, 8).reshape(1, 128)

    _deg_kernel, _edge_kernel = _sc_kernels()
    hp = _mmh(x, W1)
    deg16 = _deg_kernel(dst_p)
    g1p, dinvp = _scale(deg16.reshape(2, NP8, 128), hp)
    pre1 = _edge_kernel(g1p.reshape(N_PAD, H), src_p, dst_p)
    g2p = _layer2(dinvp, pre1.reshape(2, NP8, 128), g1p, b1t, w2bd)
    pre2 = _edge_kernel(g2p.reshape(N_PAD, H), src_p, dst_p)
    outp = _final(dinvp, pre2.reshape(2, NP8, 128), g2p, b2t)
    out = outp.reshape(NP8, 8, H).transpose(1, 0, 2).reshape(N_PAD, H)
    return out[:n, :10]


# speedup vs baseline: 1.1056x; 1.0109x over previous
import functools

import jax
import jax.numpy as jnp
from jax import lax
from jax.experimental import pallas as pl
from jax.experimental.pallas import tpu as pltpu
from jax.experimental.pallas import tpu_sc as plsc

N_NODES = 10000
N_PAD = 10240
H = 16
NW = 32
N_CHUNK = 80
CH = 128
E_PAD = NW * N_CHUNK * CH
ROWS_PER_TILE = N_PAD // 16
NP8 = N_PAD // 8

DEG_WIN = 24


def _deg_body(dst_hbm, out_hbm, idx_v, ones_v, buf_v, buf16_v, acc_sh, dsem):
    cid = lax.axis_index("c")
    sid = lax.axis_index("s")
    wid = cid * 16 + sid

    def _fill_ones(i, _):
        ones_v[pl.ds(i * 16, 16)] = jnp.ones((16,), jnp.float32)
        return 0

    lax.fori_loop(0, CH // 16, _fill_ones, 0)

    def _fill_zeros(i, _):
        buf_v[pl.ds(i * 16, 16)] = jnp.zeros((16,), jnp.float32)
        return 0

    lax.fori_loop(0, ROWS_PER_TILE // 16, _fill_zeros, 0)
    pltpu.sync_copy(buf_v, acc_sh.at[pl.ds(sid * ROWS_PER_TILE, ROWS_PER_TILE)])
    pltpu.sync_copy(dst_hbm.at[wid], idx_v)
    plsc.subcore_barrier()

    def _edges(j, _):
        @pl.when(j >= DEG_WIN)
        def _absorb():
            pltpu.make_async_copy(ones_v, acc_sh.at[idx_v.at[0]], dsem).wait()

        pltpu.async_copy(ones_v, acc_sh.at[idx_v.at[j]], dsem, add=True)
        return 0

    lax.fori_loop(0, N_CHUNK, _edges, 0)
    for _ in range(DEG_WIN):
        pltpu.make_async_copy(ones_v, acc_sh.at[idx_v.at[0]], dsem).wait()
    plsc.subcore_barrier()
    pltpu.sync_copy(acc_sh.at[pl.ds(sid * ROWS_PER_TILE, ROWS_PER_TILE)], buf_v)

    def _bcast(i, _):
        v = buf_v[pl.ds(i * 16, 16)]
        for l in range(16):
            buf16_v[i * 16 + l, :] = jnp.full((H,), v[l], jnp.float32)
        return 0

    lax.fori_loop(0, ROWS_PER_TILE // 16, _bcast, 0)
    pltpu.sync_copy(buf16_v, out_hbm.at[cid].at[pl.ds(sid * ROWS_PER_TILE, ROWS_PER_TILE)])


NSLOT = 10
GLOOK = 6


def _edge_body(g_hbm, src_hbm, dst_hbm, out_hbm, src_v, dst_v, rows_v, buf_v, acc_sh,
               isem, *sems):
    gsems = sems[:NSLOT]
    ssems = sems[NSLOT:]
    cid = lax.axis_index("c")
    sid = lax.axis_index("s")
    wid = cid * 16 + sid

    pltpu.async_copy(src_hbm.at[wid], src_v, isem)
    pltpu.async_copy(dst_hbm.at[wid], dst_v, isem)

    def _fill_zeros(i, _):
        buf_v[i, :] = jnp.zeros((H,), jnp.float32)
        return 0

    lax.fori_loop(0, ROWS_PER_TILE, _fill_zeros, 0)
    pltpu.sync_copy(buf_v, acc_sh.at[pl.ds(sid * ROWS_PER_TILE, ROWS_PER_TILE)])
    pltpu.make_async_copy(src_hbm.at[wid], src_v, isem).wait()
    pltpu.make_async_copy(dst_hbm.at[wid], dst_v, isem).wait()
    plsc.subcore_barrier()

    for b in range(GLOOK):
        pltpu.async_copy(g_hbm.at[src_v.at[b]], rows_v.at[b], gsems[b])

    def _rounds(o, _):
        for b in range(NSLOT):
            j = o * NSLOT + b
            bg = (b + GLOOK) % NSLOT

            @pl.when(j + GLOOK < N_CHUNK)
            def _refill():
                @pl.when(j >= NSLOT - GLOOK)
                def _absorb():
                    pltpu.make_async_copy(
                        rows_v.at[0], acc_sh.at[dst_v.at[0]], ssems[bg]).wait()

                pltpu.async_copy(g_hbm.at[src_v.at[j + GLOOK]], rows_v.at[bg], gsems[bg])

            pltpu.make_async_copy(g_hbm.at[src_v.at[j]], rows_v.at[b], gsems[b]).wait()
            pltpu.async_copy(rows_v.at[b], acc_sh.at[dst_v.at[j]], ssems[b], add=True)
        return 0

    lax.fori_loop(0, N_CHUNK // NSLOT, _rounds, 0)
    for b in range(NSLOT):
        pltpu.make_async_copy(rows_v.at[0], acc_sh.at[dst_v.at[0]], ssems[b]).wait()
    plsc.subcore_barrier()
    pltpu.sync_copy(acc_sh.at[pl.ds(sid * ROWS_PER_TILE, ROWS_PER_TILE)], buf_v)
    pltpu.sync_copy(buf_v, out_hbm.at[cid].at[pl.ds(sid * ROWS_PER_TILE, ROWS_PER_TILE)])


@functools.cache
def _sc_kernels():
    mesh = plsc.VectorSubcoreMesh(core_axis_name="c", subcore_axis_name="s")
    params = pltpu.CompilerParams(use_tc_tiling_on_sc=False)
    deg_kernel = pl.kernel(
        _deg_body,
        out_type=jax.ShapeDtypeStruct((2, N_PAD, H), jnp.float32),
        mesh=mesh,
        compiler_params=params,
        scratch_types=[
            pltpu.VMEM((N_CHUNK, CH), jnp.int32),
            pltpu.VMEM((CH,), jnp.float32),
            pltpu.VMEM((ROWS_PER_TILE,), jnp.float32),
            pltpu.VMEM((ROWS_PER_TILE, H), jnp.float32),
            pltpu.VMEM_SHARED((N_PAD,), jnp.float32),
            pltpu.SemaphoreType.DMA,
        ],
    )
    edge_kernel = pl.kernel(
        _edge_body,
        out_type=jax.ShapeDtypeStruct((2, N_PAD, H), jnp.float32),
        mesh=mesh,
        compiler_params=params,
        scratch_types=[
            pltpu.VMEM((N_CHUNK, CH), jnp.int32),
            pltpu.VMEM((N_CHUNK, CH), jnp.int32),
            pltpu.VMEM((NSLOT, CH, H), jnp.float32),
            pltpu.VMEM((ROWS_PER_TILE, H), jnp.float32),
            pltpu.VMEM_SHARED((N_PAD, H), jnp.float32),
            pltpu.SemaphoreType.DMA,
        ] + [pltpu.SemaphoreType.DMA] * (2 * NSLOT),
    )
    return deg_kernel, edge_kernel


def _mmh_body(x_ref, w_ref, o_ref):
    h = jnp.dot(x_ref[...], w_ref[...], preferred_element_type=jnp.float32)
    jrow = lax.broadcasted_iota(jnp.int32, (H, 128), 0)
    lcol = lax.broadcasted_iota(jnp.int32, (H, 128), 1)
    acc = jnp.zeros((NP8, 128), jnp.float32)
    for b in range(8):
        if (b + 1) * NP8 <= N_NODES:
            blk = lax.slice(h, (b * NP8, 0), ((b + 1) * NP8, H))
        else:
            blk = jnp.concatenate(
                [lax.slice(h, (b * NP8, 0), (N_NODES, H)),
                 jnp.zeros(((b + 1) * NP8 - N_NODES, H), jnp.float32)], axis=0)
        sel = (lcol == H * b + jrow).astype(jnp.float32)
        acc = acc + jnp.dot(blk, sel, preferred_element_type=jnp.float32)
    o_ref[...] = acc


def _scale_body(deg_ref, h_ref, g_ref, dinv_ref):
    d = deg_ref[...]
    dinv = lax.rsqrt(d[0] + d[1] + 1.0)
    dinv_ref[...] = dinv
    g_ref[...] = dinv * h_ref[...]


def _layer2_body(dinv_ref, pre_ref, g1_ref, b1_ref, w2_ref, o_ref):
    dinv = dinv_ref[...]
    p = pre_ref[...]
    out1 = jnp.maximum(dinv * (g1_ref[...] + p[0] + p[1]) + b1_ref[...], 0.0)
    o_ref[...] = dinv * jnp.dot(out1, w2_ref[...], preferred_element_type=jnp.float32)


def _final_body(dinv_ref, pre_ref, g2_ref, b2_ref, o_ref):
    p = pre_ref[...]
    logits = dinv_ref[...] * (g2_ref[...] + p[0] + p[1]) + b2_ref[...]
    col = lax.broadcasted_iota(jnp.int32, (NP8, 128), 1)
    xm = jnp.where(col % H < 10, logits, jnp.float32(-1e30))
    m = jnp.max(xm, axis=1, keepdims=True)
    e = jnp.where(col % H < 10, jnp.exp(logits - m), 0.0)
    r = lax.broadcasted_iota(jnp.int32, (128, 128), 0)
    c = lax.broadcasted_iota(jnp.int32, (128, 128), 1)
    ones_bd = (r // H == c // H).astype(jnp.float32)
    s = jnp.dot(e, ones_bd, preferred_element_type=jnp.float32)
    o_ref[...] = (logits - m) - jnp.log(s)


_mmh = pl.pallas_call(_mmh_body, out_shape=jax.ShapeDtypeStruct((NP8, 128), jnp.float32))
_scale = pl.pallas_call(_scale_body, out_shape=(
    jax.ShapeDtypeStruct((NP8, 128), jnp.float32),
    jax.ShapeDtypeStruct((NP8, 128), jnp.float32)))
_layer2 = pl.pallas_call(_layer2_body, out_shape=jax.ShapeDtypeStruct((NP8, 128), jnp.float32))
_final = pl.pallas_call(_final_body, out_shape=jax.ShapeDtypeStruct((NP8, 128), jnp.float32))


def kernel(x, edge_index, W1, b1, W2, b2):
    n, f = x.shape
    pad_e = E_PAD - edge_index.shape[1]
    dummy = jnp.broadcast_to(
        (N_NODES + jnp.arange(pad_e, dtype=jnp.int32) % (N_PAD - N_NODES))
        .reshape(pad_e // CH, CH), (2, pad_e // CH, CH))
    ep = jnp.concatenate([edge_index.reshape(2, -1, CH), dummy], axis=1)
    ep = (ep % NP8) * 8 + ep // NP8
    src_p = ep[0].reshape(NW, N_CHUNK, CH)
    dst_p = ep[1].reshape(NW, N_CHUNK, CH)
    w2p = jnp.concatenate([W2, jnp.zeros((W2.shape[0], H - W2.shape[1]), W2.dtype)], axis=1)
    w2bd = jax.scipy.linalg.block_diag(*([w2p] * 8))
    b1t = jnp.tile(b1, 8).reshape(1, 128)
    b2p = jnp.concatenate([b2, jnp.zeros((H - b2.shape[0],), b2.dtype)])
    b2t = jnp.tile(b2p, 8).reshape(1, 128)

    _deg_kernel, _edge_kernel = _sc_kernels()
    hp = _mmh(x, W1)
    deg16 = _deg_kernel(dst_p)
    g1p, dinvp = _scale(deg16.reshape(2, NP8, 128), hp)
    pre1 = _edge_kernel(g1p.reshape(N_PAD, H), src_p, dst_p)
    g2p = _layer2(dinvp, pre1.reshape(2, NP8, 128), g1p, b1t, w2bd)
    pre2 = _edge_kernel(g2p.reshape(N_PAD, H), src_p, dst_p)
    outp = _final(dinvp, pre2.reshape(2, NP8, 128), g2p, b2t)
    out = outp.reshape(NP8, 8, H).transpose(1, 0, 2).reshape(N_PAD, H)
    return out[:n, :10]
